# baseline (speedup 1.0000x reference)
import functools

import jax
import jax.numpy as jnp
from jax import lax
from jax.experimental import pallas as pl
from jax.experimental.pallas import tpu as pltpu
from jax.experimental.pallas import tpu_sc as plsc

HIDDEN = 128
SEQ = 200
NLANE = 16
NCHUNK = HIDDEN // NLANE
NWORK = 32
SPW = 32
ROWS = 2 * SPW
NPAIR = SEQ // 2
NBUF = 5

_GATHER_DNUMS = lax.GatherDimensionNumbers(
    offset_dims=(), collapsed_slice_dims=(0,), start_index_map=(0,))


def _shuffle(v, idx):
    return lax.gather(v, idx, _GATHER_DNUMS, (1,),
                      mode=lax.GatherScatterMode.PROMISE_IN_BOUNDS)


def _allsum(v, perms):
    for idx in perms:
        v = v + _shuffle(v, idx)
    return v


def _rsqrt(x):
    i = lax.bitcast_convert_type(x, jnp.int32)
    i = 0x5F3759DF - (i >> 1)
    y = lax.bitcast_convert_type(i, jnp.float32)
    return y * (1.5 - (0.5 * x) * y * y)


def _sc_embed(ids_hbm, pos_hbm, tok_hbm, gamma_hbm, beta_hbm, out_hbm,
              pos_v, gamma_v, beta_v, idx_v, patt_v, oidx_v, bufs,
              gsems, osems):
    nc = 2
    wid = lax.axis_index("s") * nc + lax.axis_index("c")

    pltpu.sync_copy(pos_hbm, pos_v)
    pltpu.sync_copy(gamma_hbm, gamma_v)
    pltpu.sync_copy(beta_hbm, beta_v)
    pltpu.sync_copy(ids_hbm.at[wid], idx_v)

    def drain(sem, buf):
        pltpu.make_async_copy(out_hbm.at[pl.ds(0, ROWS)], buf, sem).wait()

    def gather(j, s):
        pltpu.async_copy(tok_hbm.at[idx_v.at[j]], bufs[s], gsems[s])

    lane = lax.iota(jnp.int32, NLANE)
    perms = [(lane ^ sh).reshape(NLANE, 1) for sh in (8, 4, 2, 1)]
    g = [gamma_v[pl.ds(k * NLANE, NLANE)] for k in range(NCHUNK)]
    bt = [beta_v[pl.ds(k * NLANE, NLANE)] for k in range(NCHUNK)]

    for m in range(ROWS // NLANE):
        j = lane + (m * NLANE)
        patt_v[pl.ds(m * NLANE, NLANE)] = 200 * (j >> 1) + (j & 1)

    out_base = wid * (SPW * SEQ)

    def compute(buf, q, s):
        off = out_base + 2 * q
        for m in range(ROWS // NLANE):
            oidx_v[s, pl.ds(m * NLANE, NLANE)] = (
                patt_v[pl.ds(m * NLANE, NLANE)] + off)

        pa = [pos_v[2 * q, pl.ds(k * NLANE, NLANE)] for k in range(NCHUNK)]
        pb = [pos_v[2 * q + 1, pl.ds(k * NLANE, NLANE)]
              for k in range(NCHUNK)]

        def one_row(r, p):
            t = [buf[r, pl.ds(k * NLANE, NLANE)] + p[k]
                 for k in range(NCHUNK)]
            s1 = t[0]
            s2 = t[0] * t[0]
            for k in range(1, NCHUNK):
                s1 = s1 + t[k]
                s2 = s2 + t[k] * t[k]
            total = _allsum(s1, perms)
            totsq = _allsum(s2, perms)
            mean = total * (1.0 / HIDDEN)
            var = totsq * (1.0 / HIDDEN) - mean * mean
            rstd = _rsqrt(var + 1e-12)
            mr = mean * rstd
            for k in range(NCHUNK):
                buf[r, pl.ds(k * NLANE, NLANE)] = (
                    (t[k] * rstd - mr) * g[k] + bt[k])

        def body(db, c):
            one_row(2 * db, pa)
            one_row(2 * db + 1, pb)
            return c

        lax.fori_loop(0, SPW, body, 0)

    for s in range(NBUF - 1):
        gather(s, s)

    def iter_body(it, carry):
        for s in range(NBUF):
            q = it * NBUF + s
            s_next = (s + NBUF - 1) % NBUF

            if s == 0:
                @pl.when(it > 0)
                def _():
                    drain(osems[s_next], bufs[s_next])
                gather(q + NBUF - 1, s_next)
            else:
                @pl.when(it < (NPAIR // NBUF) - 1)
                def _():
                    drain(osems[s_next], bufs[s_next])
                    gather(q + NBUF - 1, s_next)

            drain(gsems[s], bufs[s])
            compute(bufs[s], q, s)
            pltpu.async_copy(bufs[s], out_hbm.at[oidx_v.at[s]], osems[s])
        return carry

    lax.fori_loop(0, NPAIR // NBUF, iter_body, 0)
    for s in range(NBUF):
        drain(osems[s], bufs[s])


def kernel(input_ids, token_table, pos_table, gamma, beta):
    batch, seq = input_ids.shape
    ids_prep = (input_ids.astype(jnp.int32)
                .reshape(NWORK, SPW, NPAIR, 2)
                .transpose(0, 2, 1, 3)
                .reshape(NWORK, NPAIR, ROWS))
    pos = pos_table[:seq]

    mesh = plsc.VectorSubcoreMesh(core_axis_name="c", subcore_axis_name="s")
    run = functools.partial(
        pl.kernel,
        out_type=jax.ShapeDtypeStruct((batch * seq, HIDDEN), jnp.float32),
        mesh=mesh,
        scratch_types=[
            pltpu.VMEM((SEQ, HIDDEN), jnp.float32),
            pltpu.VMEM((HIDDEN,), jnp.float32),
            pltpu.VMEM((HIDDEN,), jnp.float32),
            pltpu.VMEM((NPAIR, ROWS), jnp.int32),
            pltpu.VMEM((ROWS,), jnp.int32),
            pltpu.VMEM((NBUF, ROWS), jnp.int32),
            [pltpu.VMEM((ROWS, HIDDEN), jnp.float32)] * NBUF,
            [pltpu.SemaphoreType.DMA] * NBUF,
            [pltpu.SemaphoreType.DMA] * NBUF,
        ],
    )(_sc_embed)
    out = run(ids_prep, pos, token_table, gamma, beta)
    return out.reshape(batch, seq, HIDDEN)


# speedup vs baseline: 1.0452x; 1.0452x over previous
import functools

import jax
import jax.numpy as jnp
from jax import lax
from jax.experimental import pallas as pl
from jax.experimental.pallas import tpu as pltpu
from jax.experimental.pallas import tpu_sc as plsc

HIDDEN = 128
SEQ = 200
NLANE = 16
NCHUNK = HIDDEN // NLANE
NWORK = 32
SPW = 32
ROWS = 2 * SPW
NPAIR = SEQ // 2
NBUF = 4

_GATHER_DNUMS = lax.GatherDimensionNumbers(
    offset_dims=(), collapsed_slice_dims=(0,), start_index_map=(0,))


def _shuffle(v, idx):
    return lax.gather(v, idx, _GATHER_DNUMS, (1,),
                      mode=lax.GatherScatterMode.PROMISE_IN_BOUNDS)


def _allsum(v, perms):
    for idx in perms:
        v = v + _shuffle(v, idx)
    return v


def _rsqrt(x):
    i = lax.bitcast_convert_type(x, jnp.int32)
    i = 0x5F3759DF - (i >> 1)
    y = lax.bitcast_convert_type(i, jnp.float32)
    return y * (1.5 - (0.5 * x) * y * y)


def _sc_embed(ids_hbm, pos_hbm, tok_hbm, gamma_hbm, beta_hbm, out_hbm,
              pos_v, gamma_v, beta_v, idx_v, patt_v, oidx_v, bufs,
              gsems, osems):
    nc = 2
    wid = lax.axis_index("s") * nc + lax.axis_index("c")

    pltpu.sync_copy(pos_hbm, pos_v)
    pltpu.sync_copy(gamma_hbm, gamma_v)
    pltpu.sync_copy(beta_hbm, beta_v)
    pltpu.sync_copy(ids_hbm.at[wid], idx_v)

    def drain(sem, buf):
        pltpu.make_async_copy(out_hbm.at[pl.ds(0, ROWS)], buf, sem).wait()

    def gather(j, s):
        pltpu.async_copy(tok_hbm.at[idx_v.at[j]], bufs[s], gsems[s])

    lane = lax.iota(jnp.int32, NLANE)
    perms = [(lane ^ sh).reshape(NLANE, 1) for sh in (8, 4, 2, 1)]
    g = [gamma_v[pl.ds(k * NLANE, NLANE)] for k in range(NCHUNK)]
    bt = [beta_v[pl.ds(k * NLANE, NLANE)] for k in range(NCHUNK)]

    for m in range(ROWS // NLANE):
        j = lane + (m * NLANE)
        patt_v[pl.ds(m * NLANE, NLANE)] = 200 * (j >> 1) + (j & 1)

    out_base = wid * (SPW * SEQ)

    def compute(buf, q, s):
        off = out_base + 2 * q
        for m in range(ROWS // NLANE):
            oidx_v[s, pl.ds(m * NLANE, NLANE)] = (
                patt_v[pl.ds(m * NLANE, NLANE)] + off)

        pa = [pos_v[2 * q, pl.ds(k * NLANE, NLANE)] for k in range(NCHUNK)]
        pb = [pos_v[2 * q + 1, pl.ds(k * NLANE, NLANE)]
              for k in range(NCHUNK)]

        def one_row(r, p):
            t = [buf[r, pl.ds(k * NLANE, NLANE)] + p[k]
                 for k in range(NCHUNK)]
            a = [t[2 * i] + t[2 * i + 1] for i in range(4)]
            s1 = (a[0] + a[1]) + (a[2] + a[3])
            b = [t[2 * i] * t[2 * i] + t[2 * i + 1] * t[2 * i + 1]
                 for i in range(4)]
            s2 = (b[0] + b[1]) + (b[2] + b[3])
            total = _allsum(s1, perms)
            totsq = _allsum(s2, perms)
            mean = total * (1.0 / HIDDEN)
            var = totsq * (1.0 / HIDDEN) - mean * mean
            rstd = _rsqrt(var + 1e-12)
            mr = mean * rstd
            for k in range(NCHUNK):
                buf[r, pl.ds(k * NLANE, NLANE)] = (
                    (t[k] * rstd - mr) * g[k] + bt[k])

        def body(m, c):
            one_row(4 * m, pa)
            one_row(4 * m + 1, pb)
            one_row(4 * m + 2, pa)
            one_row(4 * m + 3, pb)
            return c

        lax.fori_loop(0, SPW // 2, body, 0)

    for s in range(NBUF - 1):
        gather(s, s)

    def iter_body(it, carry):
        for s in range(NBUF):
            q = it * NBUF + s
            s_next = (s + NBUF - 1) % NBUF

            if s == 0:
                @pl.when(it > 0)
                def _():
                    drain(osems[s_next], bufs[s_next])
                gather(q + NBUF - 1, s_next)
            else:
                @pl.when(it < (NPAIR // NBUF) - 1)
                def _():
                    drain(osems[s_next], bufs[s_next])
                    gather(q + NBUF - 1, s_next)

            drain(gsems[s], bufs[s])
            compute(bufs[s], q, s)
            pltpu.async_copy(bufs[s], out_hbm.at[oidx_v.at[s]], osems[s])
        return carry

    lax.fori_loop(0, NPAIR // NBUF, iter_body, 0)
    for s in range(NBUF):
        drain(osems[s], bufs[s])


def kernel(input_ids, token_table, pos_table, gamma, beta):
    batch, seq = input_ids.shape
    ids_prep = (input_ids.astype(jnp.int32)
                .reshape(NWORK, SPW, NPAIR, 2)
                .transpose(0, 2, 1, 3)
                .reshape(NWORK, NPAIR, ROWS))
    pos = pos_table[:seq]

    mesh = plsc.VectorSubcoreMesh(core_axis_name="c", subcore_axis_name="s")
    run = functools.partial(
        pl.kernel,
        out_type=jax.ShapeDtypeStruct((batch * seq, HIDDEN), jnp.float32),
        mesh=mesh,
        scratch_types=[
            pltpu.VMEM((SEQ, HIDDEN), jnp.float32),
            pltpu.VMEM((HIDDEN,), jnp.float32),
            pltpu.VMEM((HIDDEN,), jnp.float32),
            pltpu.VMEM((NPAIR, ROWS), jnp.int32),
            pltpu.VMEM((ROWS,), jnp.int32),
            pltpu.VMEM((NBUF, ROWS), jnp.int32),
            [pltpu.VMEM((ROWS, HIDDEN), jnp.float32)] * NBUF,
            [pltpu.SemaphoreType.DMA] * NBUF,
            [pltpu.SemaphoreType.DMA] * NBUF,
        ],
    )(_sc_embed)
    out = run(ids_prep, pos, token_table, gamma, beta)
    return out.reshape(batch, seq, HIDDEN)
